# baseline (speedup 1.0000x reference)
import functools

import jax
import jax.numpy as jnp
from jax import lax
from jax.experimental import pallas as pl
from jax.experimental.pallas import tpu as pltpu, tpu_sc as plsc

BATCH = 16384
EMBED = 32
_NC = 2
_NS = 16
_NW = _NC * _NS
_BPW = BATCH // _NW
_CHUNK = 64
_NCHUNK = _BPW // _CHUNK
_GPC = _CHUNK // 16


def _body(crow_hbm, arow_hbm, ctab_hbm, atab_hbm, cbias_hbm, abias_hbm,
          out_hbm, cidx_v, aidx_v, crows_v, arows_v, cb_v, ab_v, out_v, sems):
    wid = lax.axis_index("s") * _NC + lax.axis_index("c")
    base = wid * _BPW

    pltpu.sync_copy(crow_hbm.at[pl.ds(base, _BPW)], cidx_v)
    pltpu.sync_copy(arow_hbm.at[pl.ds(base, _BPW)], aidx_v)

    iota = lax.broadcasted_iota(jnp.int32, (16,), 0)

    def fire_chunk(k, p):
        sem = sems.at[p]

        def fire16(f, carry):
            off = k * _CHUNK + f * 16
            vc = cidx_v[pl.ds(off, 16)]
            va = aidx_v[pl.ds(off, 16)]
            for j in range(16):
                r = vc[j]
                s = va[j]
                i = f * 16 + j
                pltpu.async_copy(ctab_hbm.at[pl.ds(r, 1), :],
                                 crows_v.at[p, pl.ds(i, 1), :], sem)
                pltpu.async_copy(atab_hbm.at[pl.ds(s, 1), :],
                                 arows_v.at[p, pl.ds(i, 1), :], sem)
                pltpu.async_copy(cbias_hbm.at[pl.ds(r, 1), :],
                                 cb_v.at[p, pl.ds(i, 1), :], sem)
                pltpu.async_copy(abias_hbm.at[pl.ds(s, 1), :],
                                 ab_v.at[p, pl.ds(i, 1), :], sem)
            return carry

        lax.fori_loop(0, _GPC, fire16, None)

    def wait_chunk(p):
        sem = sems.at[p]
        rows_src = ctab_hbm.at[pl.ds(0, _CHUNK), :]
        bias_src = cbias_hbm.at[pl.ds(0, _CHUNK), :]
        pltpu.make_async_copy(rows_src, crows_v.at[0], sem).wait()
        pltpu.make_async_copy(rows_src, arows_v.at[0], sem).wait()
        pltpu.make_async_copy(bias_src, cb_v.at[0], sem).wait()
        pltpu.make_async_copy(bias_src, ab_v.at[0], sem).wait()

    def compute_chunk(k, p):
        pv = iota * 0 + p
        zerov = iota * 0

        def group(g, carry):
            rowv = iota + g * 16
            acc = (plsc.load_gather(cb_v, [pv, rowv, zerov])
                   + plsc.load_gather(ab_v, [pv, rowv, zerov]))
            for j in range(EMBED):
                colv = iota * 0 + j
                cv = plsc.load_gather(crows_v, [pv, rowv, colv])
                av = plsc.load_gather(arows_v, [pv, rowv, colv])
                acc = acc + cv * av
            out_v[pl.ds(k * _CHUNK + g * 16, 16)] = 1.0 / (1.0 + jnp.exp(-acc))
            return carry

        lax.fori_loop(0, _GPC, group, None)

    fire_chunk(0, 0)

    def chunk_body(k, carry):
        p = k % 2

        @pl.when(k + 1 < _NCHUNK)
        def _():
            fire_chunk(k + 1, (k + 1) % 2)

        wait_chunk(p)
        compute_chunk(k, p)
        return carry

    lax.fori_loop(0, _NCHUNK, chunk_body, None)

    pltpu.sync_copy(out_v, out_hbm.at[pl.ds(base, _BPW)])


@jax.jit
def _hm_model(crow, arow, ctab, atab, cbias, abias):
    mesh = plsc.VectorSubcoreMesh(core_axis_name="c", subcore_axis_name="s")
    kfn = functools.partial(
        pl.kernel,
        mesh=mesh,
        compiler_params=pltpu.CompilerParams(needs_layout_passes=False),
        out_type=jax.ShapeDtypeStruct((BATCH,), jnp.float32),
        scratch_types=[
            pltpu.VMEM((_BPW,), jnp.int32),
            pltpu.VMEM((_BPW,), jnp.int32),
            pltpu.VMEM((2, _CHUNK, EMBED), jnp.float32),
            pltpu.VMEM((2, _CHUNK, EMBED), jnp.float32),
            pltpu.VMEM((2, _CHUNK, 1), jnp.float32),
            pltpu.VMEM((2, _CHUNK, 1), jnp.float32),
            pltpu.VMEM((_BPW,), jnp.float32),
            pltpu.SemaphoreType.DMA((2,)),
        ],
    )(_body)
    return kfn(crow, arow, ctab, atab, cbias, abias)


def kernel(customer_row, article_row, customer_table, article_table,
           customer_bias, article_bias):
    out = _hm_model(customer_row.astype(jnp.int32), article_row.astype(jnp.int32),
                    customer_table, article_table, customer_bias, article_bias)
    return out.reshape(BATCH, 1)
